# Initial kernel scaffold; baseline (speedup 1.0000x reference)
import jax
import jax.numpy as jnp
from jax.experimental import pallas as pl


def kernel(atomic_numbers, edge_index, edge_length, hopping_param, overlap_param, onsite_param):
    raise NotImplementedError("write your pallas kernel here")


# speedup vs baseline: 38.1534x; 38.1534x over previous
import functools

import jax
import jax.numpy as jnp
from jax import lax
from jax.experimental import pallas as pl
from jax.experimental.pallas import tpu as pltpu
from jax.experimental.pallas import tpu_sc as plsc

_RC = 5.0
_W = 1.0
_BL0 = 2.35
_BL1 = 1.54
_EQ_ORB = (1.0, 0.0, 1.0, 1.0)

_NC = 2
_NS = 16
_NW = _NC * _NS


def _sc_gather(anum, idx_flat):
    total = idx_flat.shape[0]
    per_w = total // _NW
    ch = 8000
    n_it = per_w // ch
    mesh = plsc.VectorSubcoreMesh(core_axis_name="c", subcore_axis_name="s")

    @functools.partial(
        pl.kernel,
        out_type=jax.ShapeDtypeStruct((total,), jnp.int32),
        mesh=mesh,
        scratch_types=[
            pltpu.VMEM((ch,), jnp.int32),
            pltpu.VMEM((ch,), jnp.int32),
            pltpu.SemaphoreType.DMA,
        ],
    )
    def k(anum_hbm, idx_hbm, out_hbm, idx_v, typ_v, sem):
        wid = lax.axis_index("s") * _NC + lax.axis_index("c")
        base = wid * per_w

        def body(i, carry):
            off = pl.multiple_of(base + i * ch, 8)
            pltpu.sync_copy(idx_hbm.at[pl.ds(off, ch)], idx_v)
            pltpu.async_copy(anum_hbm.at[idx_v], typ_v, sem).wait()
            pltpu.sync_copy(typ_v, out_hbm.at[pl.ds(off, ch)])
            return carry

        lax.fori_loop(0, n_it, body, 0)

    return k(anum, idx_flat)


def _edge_body(ti_ref, tj_ref, rij_ref, ha1_ref, ha2_ref, oa1_ref, oa2_ref,
               eq_ref, s_ref, feat_ref, ov_ref):
    r = rij_ref.shape[0]
    tif = ti_ref[0].astype(jnp.float32)
    tjf = tj_ref[0].astype(jnp.float32)
    rij = rij_ref[...]
    bond = 2.0 * tif + tjf
    eq = (tif == tjf).astype(jnp.float32)
    r0 = 0.5 * (2.0 * _BL0 + (_BL1 - _BL0) * (tif + tjf))
    lr = jnp.log(r0 / rij)
    fc = 1.0 / (1.0 + jnp.exp((rij - _RC + 5.0 * _W) / _W))
    packed = jnp.concatenate([bond, eq, lr, fc], axis=0)
    ex = jnp.dot(packed, s_ref[...], preferred_element_type=jnp.float32)
    bond4 = ex[0:r]
    eq4 = ex[r:2 * r]
    lr4 = ex[2 * r:3 * r]
    fc4 = ex[3 * r:4 * r]
    m0 = (bond4 == 0.0).astype(jnp.float32)
    m1 = (bond4 == 1.0).astype(jnp.float32)
    m2 = (bond4 == 2.0).astype(jnp.float32)
    m3 = (bond4 == 3.0).astype(jnp.float32)

    def sel(tab_ref):
        return (m0 * tab_ref[0:1, :] + m1 * tab_ref[1:2, :]
                + m2 * tab_ref[2:3, :] + m3 * tab_ref[3:4, :])

    feat_ref[...] = sel(ha1_ref) * jnp.exp(sel(ha2_ref) * lr4) * fc4
    ov_ref[...] = (sel(oa1_ref) * jnp.exp(sel(oa2_ref) * lr4)
                   + eq4 * eq_ref[0:1, :]) * fc4


def _tc_edges(types3, rij2, ha1, ha2, oa1, oa2, eqrow, smat):
    rows = rij2.shape[0]
    blk = 2000
    grid = rows // blk
    tiny = pl.BlockSpec(None, lambda i: (0,) * 2)
    feat, ov = pl.pallas_call(
        _edge_body,
        grid=(grid,),
        in_specs=[
            pl.BlockSpec((1, blk, 32), lambda i: (0, i, 0)),
            pl.BlockSpec((1, blk, 32), lambda i: (1, i, 0)),
            pl.BlockSpec((blk, 32), lambda i: (i, 0)),
            tiny, tiny, tiny, tiny, tiny, tiny,
        ],
        out_specs=[
            pl.BlockSpec((blk, 128), lambda i: (i, 0)),
            pl.BlockSpec((blk, 128), lambda i: (i, 0)),
        ],
        out_shape=[
            jax.ShapeDtypeStruct((rows, 128), jnp.float32),
            jax.ShapeDtypeStruct((rows, 128), jnp.float32),
        ],
    )(types3, types3, rij2, ha1, ha2, oa1, oa2, eqrow, smat)
    return feat, ov


def _node_body(t_ref, on0_ref, don_ref, s2_ref, out_ref):
    tf = t_ref[...].astype(jnp.float32)
    t2 = jnp.dot(tf, s2_ref[...], preferred_element_type=jnp.float32)
    out_ref[...] = on0_ref[0:1, :] + t2 * don_ref[0:1, :]


def _tc_nodes(t2d, on0, don, s2):
    rows = t2d.shape[0]
    return pl.pallas_call(
        _node_body,
        out_shape=jax.ShapeDtypeStruct((rows, 32), jnp.float32),
    )(t2d, on0, don, s2)


def kernel(atomic_numbers, edge_index, edge_length, hopping_param,
           overlap_param, onsite_param):
    n = atomic_numbers.shape[0]
    e = edge_index.shape[1]
    rows = e // 32

    idx_flat = edge_index.reshape(-1)
    types_flat = _sc_gather(atomic_numbers, idx_flat)
    types3 = types_flat.reshape(2, rows, 32)
    rij2 = edge_length.reshape(rows, 32)

    ha1 = jnp.tile(hopping_param[:, :, 0], (1, 32))
    ha2 = jnp.tile(1.0 + jnp.abs(hopping_param[:, :, 1]), (1, 32))
    oa1 = jnp.tile(overlap_param[:, :, 0], (1, 32))
    oa2 = jnp.tile(1.0 + jnp.abs(overlap_param[:, :, 1]), (1, 32))
    eqrow = jnp.tile(jnp.asarray(_EQ_ORB, jnp.float32), (32,))[None, :]

    lane = jnp.arange(128)
    smat = (lane[None, :] // 4 == jnp.arange(32)[:, None]).astype(jnp.float32)
    lane32 = jnp.arange(32)
    s2 = (lane32[None, :] // 2 == jnp.arange(16)[:, None]).astype(jnp.float32)

    feat, ov = _tc_edges(types3, rij2, ha1, ha2, oa1, oa2, eqrow, smat)
    edge_features = feat.reshape(e, 4)
    edge_overlap = ov.reshape(e, 4)

    on0 = jnp.tile(onsite_param[0, :, 0], (16,))[None, :]
    don = jnp.tile(onsite_param[1, :, 0] - onsite_param[0, :, 0], (16,))[None, :]
    node_features = _tc_nodes(atomic_numbers.reshape(n // 16, 16), on0, don,
                              s2).reshape(n, 2)

    return edge_features, edge_overlap, node_features


# speedup vs baseline: 77.7558x; 2.0380x over previous
import functools
import math

import jax
import jax.numpy as jnp
from jax import lax
from jax.experimental import pallas as pl
from jax.experimental.pallas import tpu as pltpu
from jax.experimental.pallas import tpu_sc as plsc

_RC = 5.0
_W = 1.0
_BL0 = 2.35
_BL1 = 1.54
_EQ_ORB = (1.0, 0.0, 1.0, 1.0)

_NC = 2
_NS = 16
_NW = _NC * _NS


def _sc_gather(anum, idx_flat):
    total = idx_flat.shape[0]
    per_w = total // _NW
    ch = 8000
    n_it = per_w // ch
    mesh = plsc.VectorSubcoreMesh(core_axis_name="c", subcore_axis_name="s")

    @functools.partial(
        pl.kernel,
        out_type=jax.ShapeDtypeStruct((total,), jnp.int32),
        mesh=mesh,
        scratch_types=[
            pltpu.VMEM((ch,), jnp.int32),
            pltpu.VMEM((ch,), jnp.int32),
            pltpu.SemaphoreType.DMA,
        ],
    )
    def k(anum_hbm, idx_hbm, out_hbm, idx_v, typ_v, sem):
        wid = lax.axis_index("s") * _NC + lax.axis_index("c")
        base = wid * per_w

        def body(i, carry):
            off = pl.multiple_of(base + i * ch, 8)
            pltpu.sync_copy(idx_hbm.at[pl.ds(off, ch)], idx_v)
            pltpu.async_copy(anum_hbm.at[idx_v], typ_v, sem).wait()
            pltpu.sync_copy(typ_v, out_hbm.at[pl.ds(off, ch)])
            return carry

        lax.fori_loop(0, n_it, body, 0)

    return k(anum, idx_flat)


def _edge_body(ti_ref, tj_ref, rij_ref, hp_ref, op_ref, feat_ref, ov_ref):
    tif = ti_ref[:, 0, 0, :].astype(jnp.float32)
    tjf = tj_ref[:, 0, 0, :].astype(jnp.float32)
    rij = rij_ref[...]
    m0 = ((1.0 - tif) * (1.0 - tjf))
    m1 = ((1.0 - tif) * tjf)
    m2 = (tif * (1.0 - tjf))
    m3 = (tif * tjf)
    eq = m0 + m3
    lg00 = math.log(_BL0)
    lg01 = math.log(0.5 * (_BL0 + _BL1))
    lg11 = math.log(_BL1)
    lr = (m0 * lg00 + (m1 + m2) * lg01 + m3 * lg11) - jnp.log(rij)
    fc = 1.0 / (1.0 + jnp.exp((rij - _RC + 5.0 * _W) / _W))
    eqfc = eq * fc

    def sel(tab_ref, q, p, absval):
        vals = []
        for b in range(4):
            v = tab_ref[b, q, p]
            vals.append(jnp.abs(v) if absval else v)
        return (m0 * vals[0] + m1 * vals[1] + m2 * vals[2] + m3 * vals[3])

    for q in range(4):
        a1 = sel(hp_ref, q, 0, False)
        a2 = sel(hp_ref, q, 1, True)
        feat_ref[:, q * 128:(q + 1) * 128] = a1 * jnp.exp((1.0 + a2) * lr) * fc
        b1 = sel(op_ref, q, 0, False)
        b2 = sel(op_ref, q, 1, True)
        ovq = b1 * jnp.exp((1.0 + b2) * lr) * fc
        if _EQ_ORB[q] != 0.0:
            ovq = ovq + _EQ_ORB[q] * eqfc
        ov_ref[:, q * 128:(q + 1) * 128] = ovq


def _tc_edges(types4, rij2, hp, op):
    rows = rij2.shape[0]
    blk = 1000
    grid = rows // blk
    tiny = pl.BlockSpec(None, lambda i: (0,) * 3)
    feat, ov = pl.pallas_call(
        _edge_body,
        grid=(grid,),
        in_specs=[
            pl.BlockSpec((blk, 1, 1, 128), lambda i: (i, 0, 0, 0)),
            pl.BlockSpec((blk, 1, 1, 128), lambda i: (i, 1, 0, 0)),
            pl.BlockSpec((blk, 128), lambda i: (i, 0)),
            tiny, tiny,
        ],
        out_specs=[
            pl.BlockSpec((blk, 512), lambda i: (i, 0)),
            pl.BlockSpec((blk, 512), lambda i: (i, 0)),
        ],
        out_shape=[
            jax.ShapeDtypeStruct((rows, 512), jnp.float32),
            jax.ShapeDtypeStruct((rows, 512), jnp.float32),
        ],
    )(types4, types4, rij2, hp, op)
    return feat, ov


def _node_body(t_ref, on0_ref, don_ref, s2_ref, out_ref):
    tf = t_ref[...].astype(jnp.float32)
    t2 = jnp.dot(tf, s2_ref[...], preferred_element_type=jnp.float32)
    out_ref[...] = on0_ref[0:1, :] + t2 * don_ref[0:1, :]


def _tc_nodes(t2d, on0, don, s2):
    rows = t2d.shape[0]
    return pl.pallas_call(
        _node_body,
        out_shape=jax.ShapeDtypeStruct((rows, 32), jnp.float32),
    )(t2d, on0, don, s2)


def kernel(atomic_numbers, edge_index, edge_length, hopping_param,
           overlap_param, onsite_param):
    n = atomic_numbers.shape[0]
    e = edge_index.shape[1]
    rows = e // 128

    idx_phys = edge_index.reshape(2, rows, 128).transpose(1, 0, 2).reshape(-1)
    types_flat = _sc_gather(atomic_numbers, idx_phys)
    types4 = types_flat.reshape(rows, 2, 1, 128)
    rij2 = edge_length.reshape(rows, 128)

    feat, ov = _tc_edges(types4, rij2, hopping_param, overlap_param)
    edge_features = feat.reshape(rows, 4, 128).transpose(0, 2, 1).reshape(e, 4)
    edge_overlap = ov.reshape(rows, 4, 128).transpose(0, 2, 1).reshape(e, 4)

    lane32 = jnp.arange(32)
    s2 = (lane32[None, :] // 2 == jnp.arange(16)[:, None]).astype(jnp.float32)
    on0 = jnp.tile(onsite_param[0, :, 0], (16,))[None, :]
    don = jnp.tile(onsite_param[1, :, 0] - onsite_param[0, :, 0], (16,))[None, :]
    node_features = _tc_nodes(atomic_numbers.reshape(n // 16, 16), on0, don,
                              s2).reshape(n, 2)

    return edge_features, edge_overlap, node_features


# speedup vs baseline: 470.0294x; 6.0449x over previous
import functools
import math

import jax
import jax.numpy as jnp
from jax import lax
from jax.experimental import pallas as pl
from jax.experimental.pallas import tpu as pltpu
from jax.experimental.pallas import tpu_sc as plsc

_RC = 5.0
_W = 1.0
_BL0 = 2.35
_BL1 = 1.54
_EQ_ORB = (1.0, 0.0, 1.0, 1.0)

_NC = 2
_NS = 16
_NW = _NC * _NS

_RUNS_PER_CHUNK = 25
_EDGES_PER_CHUNK = _RUNS_PER_CHUNK * 128
_IDX_PER_CHUNK = 2 * _EDGES_PER_CHUNK


def _sc_bonds(anum, idx_phys):
    n = anum.shape[0]
    total = idx_phys.shape[0]
    edges = total // 2
    n_chunks = edges // _EDGES_PER_CHUNK
    groups = _EDGES_PER_CHUNK // 16
    mesh = plsc.VectorSubcoreMesh(core_axis_name="c", subcore_axis_name="s")

    @functools.partial(
        pl.kernel,
        out_type=jax.ShapeDtypeStruct((edges,), jnp.int32),
        mesh=mesh,
        compiler_params=pltpu.CompilerParams(needs_layout_passes=False),
        scratch_types=[
            pltpu.VMEM((n,), jnp.int32),
            pltpu.VMEM((_IDX_PER_CHUNK,), jnp.int32),
            pltpu.VMEM((_EDGES_PER_CHUNK,), jnp.int32),
        ],
    )
    def k(anum_hbm, idx_hbm, out_hbm, anum_v, idx_v, bond_v):
        wid = lax.axis_index("s") * _NC + lax.axis_index("c")
        pltpu.sync_copy(anum_hbm, anum_v)
        c_lo = (n_chunks * wid) // _NW
        c_hi = (n_chunks * (wid + 1)) // _NW

        def chunk_body(c, carry):
            off_i = pl.multiple_of(c * _IDX_PER_CHUNK, 8)
            pltpu.sync_copy(idx_hbm.at[pl.ds(off_i, _IDX_PER_CHUNK)], idx_v)

            def grp(g, cc):
                base = ((g >> 3) << 8) + ((g & 7) << 4)
                si = idx_v[pl.ds(base, 16)]
                di = idx_v[pl.ds(base + 128, 16)]
                ti = plsc.load_gather(anum_v, [si])
                tj = plsc.load_gather(anum_v, [di])
                bond_v[pl.ds(g << 4, 16)] = (ti << 1) + tj
                return cc

            lax.fori_loop(0, groups, grp, 0)
            off_o = pl.multiple_of(c * _EDGES_PER_CHUNK, 8)
            pltpu.sync_copy(bond_v, out_hbm.at[pl.ds(off_o, _EDGES_PER_CHUNK)])
            return carry

        lax.fori_loop(c_lo, c_hi, chunk_body, 0)

    return k(anum, idx_phys)


def _edge_body(bond_ref, rij_ref, hp_ref, op_ref, feat_ref, ov_ref):
    bond = bond_ref[...]
    m0 = bond == 0
    m1 = bond == 1
    m2 = bond == 2
    rij = rij_ref[...]

    def sel(s0, s1, s2, s3):
        return jnp.where(m0, s0, jnp.where(m1, s1, jnp.where(m2, s2, s3)))

    lg00 = math.log(_BL0)
    lg01 = math.log(0.5 * (_BL0 + _BL1))
    lg11 = math.log(_BL1)
    lr = sel(lg00, lg01, lg01, lg11) - jnp.log(rij)
    fc = 1.0 / (1.0 + jnp.exp((rij - _RC + 5.0 * _W) / _W))
    eqfc = jnp.where(m0 | (bond == 3), fc, 0.0)

    def tab(ref, q, p, absval):
        vals = [ref[b, q, p] for b in range(4)]
        if absval:
            vals = [jnp.abs(v) for v in vals]
        return sel(*vals)

    for q in range(4):
        a1 = tab(hp_ref, q, 0, False)
        a2 = tab(hp_ref, q, 1, True)
        feat_ref[:, q * 128:(q + 1) * 128] = a1 * jnp.exp((1.0 + a2) * lr) * fc
        b1 = tab(op_ref, q, 0, False)
        b2 = tab(op_ref, q, 1, True)
        ovq = b1 * jnp.exp((1.0 + b2) * lr) * fc
        if _EQ_ORB[q] != 0.0:
            ovq = ovq + _EQ_ORB[q] * eqfc
        ov_ref[:, q * 128:(q + 1) * 128] = ovq


def _tc_edges(bond2, rij2, hp, op):
    rows = rij2.shape[0]
    blk = 1000
    grid = rows // blk
    tiny = pl.BlockSpec(None, lambda i: (0,) * 3)
    feat, ov = pl.pallas_call(
        _edge_body,
        grid=(grid,),
        in_specs=[
            pl.BlockSpec((blk, 128), lambda i: (i, 0)),
            pl.BlockSpec((blk, 128), lambda i: (i, 0)),
            tiny, tiny,
        ],
        out_specs=[
            pl.BlockSpec((blk, 512), lambda i: (i, 0)),
            pl.BlockSpec((blk, 512), lambda i: (i, 0)),
        ],
        out_shape=[
            jax.ShapeDtypeStruct((rows, 512), jnp.float32),
            jax.ShapeDtypeStruct((rows, 512), jnp.float32),
        ],
    )(bond2, rij2, hp, op)
    return feat, ov


def _node_body(t_ref, on0_ref, don_ref, s2_ref, out_ref):
    tf = t_ref[...].astype(jnp.float32)
    t2 = jnp.dot(tf, s2_ref[...], preferred_element_type=jnp.float32)
    out_ref[...] = on0_ref[0:1, :] + t2 * don_ref[0:1, :]


def _tc_nodes(t2d, on0, don, s2):
    rows = t2d.shape[0]
    return pl.pallas_call(
        _node_body,
        out_shape=jax.ShapeDtypeStruct((rows, 32), jnp.float32),
    )(t2d, on0, don, s2)


def kernel(atomic_numbers, edge_index, edge_length, hopping_param,
           overlap_param, onsite_param):
    n = atomic_numbers.shape[0]
    e = edge_index.shape[1]
    rows = e // 128

    idx_phys = edge_index.reshape(2, rows, 128).transpose(1, 0, 2).reshape(-1)
    bond = _sc_bonds(atomic_numbers, idx_phys)
    bond2 = bond.reshape(rows, 128)
    rij2 = edge_length.reshape(rows, 128)

    feat, ov = _tc_edges(bond2, rij2, hopping_param, overlap_param)
    edge_features = feat.reshape(rows, 4, 128).transpose(0, 2, 1).reshape(e, 4)
    edge_overlap = ov.reshape(rows, 4, 128).transpose(0, 2, 1).reshape(e, 4)

    lane32 = jnp.arange(32)
    s2 = (lane32[None, :] // 2 == jnp.arange(16)[:, None]).astype(jnp.float32)
    on0 = jnp.tile(onsite_param[0, :, 0], (16,))[None, :]
    don = jnp.tile(onsite_param[1, :, 0] - onsite_param[0, :, 0], (16,))[None, :]
    node_features = _tc_nodes(atomic_numbers.reshape(n // 16, 16), on0, don,
                              s2).reshape(n, 2)

    return edge_features, edge_overlap, node_features
